# baseline (speedup 1.0000x reference)
import functools

import jax
import jax.numpy as jnp
from jax import lax
from jax.experimental import pallas as pl
from jax.experimental.pallas import tpu as pltpu
from jax.experimental.pallas import tpu_sc as plsc

B, D, H, W = 4, 64, 384, 384
N = H * W
MAX_ID = 5
NID = 8
DELTA_V = 0.5
DELTA_D = 3.0
ALPHA, BETA, GAMMA = 1.0, 1.0, 0.001

NC, NS, L = 2, 16, 16
NW = NC * NS
WPB = NW // B
NPW = N // WPB
CHUNK = 1024
NCHUNK = NPW // CHUNK
GROUPS = CHUNK // L

_mesh = lambda: plsc.VectorSubcoreMesh(core_axis_name="c", subcore_axis_name="s")
_SC_PARAMS = pltpu.CompilerParams(needs_layout_passes=False,
                                  use_tc_tiling_on_sc=False)


def _worker_id():
    return lax.axis_index("s") * NC + lax.axis_index("c")


def _pass1(emb, msk, zsum, zcnt):
    @functools.partial(
        pl.kernel,
        out_type=[
            jax.ShapeDtypeStruct((NW, NID, D), jnp.float32),
            jax.ShapeDtypeStruct((NW, L), jnp.float32),
        ],
        mesh=_mesh(),
        scratch_types=[
            pltpu.VMEM((D, CHUNK), jnp.float32),
            pltpu.VMEM((CHUNK,), jnp.int32),
            pltpu.VMEM((NID, D, L), jnp.float32),
            pltpu.VMEM((L, L), jnp.float32),
            pltpu.VMEM((NID, D), jnp.float32),
            pltpu.VMEM((L,), jnp.float32),
        ],
        compiler_params=_SC_PARAMS,
    )
    def k1(emb_hbm, msk_hbm, zsum_hbm, zcnt_hbm, sums_out, cnts_out,
           ebuf, ibuf, acc, cacc, sums_v, cnts_v):
        wid = _worker_id()
        b = wid // WPB
        base = (wid % WPB) * NPW

        pltpu.sync_copy(zsum_hbm, acc)
        pltpu.sync_copy(zcnt_hbm, cacc)

        lane = lax.broadcasted_iota(jnp.int32, (L,), 0)
        ones = jnp.ones((L,), jnp.float32)

        def chunk_body(k, carry):
            off = base + k * CHUNK
            pltpu.sync_copy(emb_hbm.at[b, :, pl.ds(off, CHUNK)], ebuf)
            pltpu.sync_copy(msk_hbm.at[b, pl.ds(off, CHUNK)], ibuf)

            def group_body(g, c2):
                p0 = pl.multiple_of(g * L, L)
                ids = ibuf[pl.ds(p0, L)]
                plsc.addupdate_scatter(cacc, [ids, lane], ones)
                for d in range(D):
                    v = ebuf[d, pl.ds(p0, L)]
                    dv = jnp.full((L,), d, jnp.int32)
                    plsc.addupdate_scatter(acc, [ids, dv, lane], v)
                return c2

            return lax.fori_loop(0, GROUPS, group_body, carry)

        lax.fori_loop(0, NCHUNK, chunk_body, 0)

        for i in range(NID):
            iv = jnp.full((L,), i, jnp.int32)
            for d0 in range(0, D, L):
                w = jnp.zeros((L,), jnp.float32)
                for l in range(L):
                    w = w + plsc.load_gather(
                        acc, [iv, d0 + lane, jnp.full((L,), l, jnp.int32)])
                sums_v[i, pl.ds(d0, L)] = w
        cw = jnp.zeros((L,), jnp.float32)
        for l in range(L):
            cw = cw + plsc.load_gather(cacc, [lane, jnp.full((L,), l, jnp.int32)])
        cnts_v[...] = cw

        pltpu.sync_copy(sums_v, sums_out.at[wid])
        pltpu.sync_copy(cnts_v, cnts_out.at[wid])

    return k1(emb, msk, zsum, zcnt)


def _finalize_centers(sums_parts, cnt_parts):
    def body(parts_ref, cnts_ref, centers_ref, wtab_ref, misc_ref):
        parts = parts_ref[...]
        cnts = cnts_ref[...]
        idv = lax.broadcasted_iota(jnp.int32, (NID,), 0)
        valid = (idv >= 1) & (idv <= MAX_ID)
        m8 = lax.broadcasted_iota(jnp.int32, (NID,), 0)
        for b in range(B):
            sums = jnp.sum(parts[b * WPB:(b + 1) * WPB], axis=0)
            cnt = jnp.sum(cnts[b * WPB:(b + 1) * WPB], axis=0)[:NID]
            pres = jnp.where(valid & (cnt > 0), 1.0, 0.0)
            safe = jnp.maximum(cnt, 1.0)
            centers = sums / safe[:, None]
            num_inst = jnp.sum(pres)
            wtab = pres / safe
            ld = jnp.float32(0.0)
            for i in range(1, MAX_ID + 1):
                for j in range(i + 1, MAX_ID + 1):
                    d2 = jnp.sum((centers[i] - centers[j]) ** 2) + 1e-12
                    hinge = jnp.maximum(2.0 * DELTA_D - jnp.sqrt(d2), 0.0)
                    ld = ld + pres[i] * pres[j] * hinge * hinge
            npairs = num_inst * (num_inst - 1.0) * 0.5
            ld = jnp.where(num_inst > 1.0, ld / jnp.maximum(npairs, 1.0), ld)
            lr = jnp.sum(pres * jnp.sqrt(jnp.sum(centers ** 2, axis=1) + 1e-12))
            lr = lr / jnp.maximum(num_inst, 1.0)
            has = (num_inst > 0).astype(jnp.float32)
            centers_ref[b] = centers
            wtab_ref[b] = wtab
            misc_ref[b] = (jnp.where(m8 == 0, ld, 0.0)
                           + jnp.where(m8 == 1, lr, 0.0)
                           + jnp.where(m8 == 2, num_inst, 0.0)
                           + jnp.where(m8 == 3, has, 0.0))

    return pl.pallas_call(
        body,
        out_shape=[
            jax.ShapeDtypeStruct((B, NID, D), jnp.float32),
            jax.ShapeDtypeStruct((B, NID), jnp.float32),
            jax.ShapeDtypeStruct((B, NID), jnp.float32),
        ],
    )(sums_parts, cnt_parts)


def _pass2(emb, msk, centers, wtab):
    @functools.partial(
        pl.kernel,
        out_type=jax.ShapeDtypeStruct((NW, L), jnp.float32),
        mesh=_mesh(),
        scratch_types=[
            pltpu.VMEM((D, CHUNK), jnp.float32),
            pltpu.VMEM((CHUNK,), jnp.int32),
            pltpu.VMEM((NID, D), jnp.float32),
            pltpu.VMEM((NID,), jnp.float32),
            pltpu.VMEM((L,), jnp.float32),
        ],
        compiler_params=_SC_PARAMS,
    )
    def k3(emb_hbm, msk_hbm, cen_hbm, wtab_hbm, out_hbm,
           ebuf, ibuf, cen_v, w_v, acc_v):
        wid = _worker_id()
        b = wid // WPB
        base = (wid % WPB) * NPW

        pltpu.sync_copy(cen_hbm.at[b], cen_v)
        pltpu.sync_copy(wtab_hbm.at[b], w_v)

        def chunk_body(k, acc):
            off = base + k * CHUNK
            pltpu.sync_copy(emb_hbm.at[b, :, pl.ds(off, CHUNK)], ebuf)
            pltpu.sync_copy(msk_hbm.at[b, pl.ds(off, CHUNK)], ibuf)

            def group_body(g, acc2):
                p0 = pl.multiple_of(g * L, L)
                ids = ibuf[pl.ds(p0, L)]
                dsq = jnp.full((L,), 1e-12, jnp.float32)
                for d in range(D):
                    v = ebuf[d, pl.ds(p0, L)]
                    c = plsc.load_gather(cen_v, [ids, jnp.full((L,), d, jnp.int32)])
                    diff = v - c
                    dsq = dsq + diff * diff
                y = plsc.bitcast(
                    jnp.int32(0x5F3759DF) - (plsc.bitcast(dsq, jnp.int32) >> 1),
                    jnp.float32)
                for _ in range(3):
                    y = y * (1.5 - 0.5 * dsq * y * y)
                dist = dsq * y
                hinge = jnp.maximum(dist - DELTA_V, 0.0)
                wgt = plsc.load_gather(w_v, [ids])
                return acc2 + hinge * hinge * wgt

            return lax.fori_loop(0, GROUPS, group_body, acc)

        acc = lax.fori_loop(0, NCHUNK, chunk_body, jnp.zeros((L,), jnp.float32))
        acc_v[...] = acc
        pltpu.sync_copy(acc_v, out_hbm.at[wid])

    return k3(emb, msk, centers, wtab)


def _combine(lv_parts, misc):
    def body(lv_ref, misc_ref, out_ref):
        lv = lv_ref[...]
        has = jnp.stack([misc_ref[b, 3] for b in range(B)])
        denom = jnp.maximum(jnp.sum(has), 1.0)
        loss_var = jnp.float32(0.0)
        loss_dist = jnp.float32(0.0)
        loss_reg = jnp.float32(0.0)
        for b in range(B):
            s = jnp.sum(lv[b * WPB:(b + 1) * WPB])
            lv_b = s / jnp.maximum(misc_ref[b, 2], 1.0)
            loss_var = loss_var + lv_b * misc_ref[b, 3]
            loss_dist = loss_dist + misc_ref[b, 0] * misc_ref[b, 3]
            loss_reg = loss_reg + misc_ref[b, 1] * misc_ref[b, 3]
        loss_var = loss_var / denom
        loss_dist = loss_dist / denom
        loss_reg = loss_reg / denom
        total = ALPHA * loss_var + BETA * loss_dist + GAMMA * loss_reg
        m8 = lax.broadcasted_iota(jnp.int32, (NID,), 0)
        out_ref[...] = (jnp.where(m8 == 0, total, 0.0)
                        + jnp.where(m8 == 1, loss_var, 0.0)
                        + jnp.where(m8 == 2, loss_dist, 0.0)
                        + jnp.where(m8 == 3, loss_reg, 0.0))

    return pl.pallas_call(
        body,
        out_shape=jax.ShapeDtypeStruct((NID,), jnp.float32),
    )(lv_parts, misc)


def kernel(embedding, instance_mask):
    emb = embedding.reshape(B, D, N)
    msk = instance_mask.reshape(B, N).astype(jnp.int32)
    zsum = jnp.zeros((NID, D, L), jnp.float32)
    zcnt = jnp.zeros((L, L), jnp.float32)

    sums_parts, cnt_parts = _pass1(emb, msk, zsum, zcnt)
    centers, wtab, misc = _finalize_centers(sums_parts, cnt_parts)
    lv_parts = _pass2(emb, msk, centers, wtab)
    out = _combine(lv_parts, misc)
    return (out[0], out[1], out[2], out[3])
